# baseline (speedup 1.0000x reference)
import functools
import math

import jax
import jax.numpy as jnp
import numpy as np
from jax import lax
from jax.experimental import pallas as pl
from jax.experimental.pallas import tpu as pltpu
from jax.experimental.pallas import tpu_sc as plsc

_NC = 2
_NS = 16
_NW = _NC * _NS
_IDX_CHUNK = 128


def _sc_gather(table, idx2d, n_rows, d):
    chunks_per_w = n_rows // _IDX_CHUNK // _NW
    rows_per_w = chunks_per_w * _IDX_CHUNK
    bn, s = idx2d.shape
    w_per_row = _NW // bn
    mesh = plsc.VectorSubcoreMesh(core_axis_name="c", subcore_axis_name="s")

    @functools.partial(
        pl.kernel,
        mesh=mesh,
        out_type=jax.ShapeDtypeStruct((n_rows, d), jnp.float32),
        scratch_types=[
            pltpu.VMEM((chunks_per_w, _IDX_CHUNK), jnp.int32),
            pltpu.VMEM((rows_per_w, d), jnp.float32),
        ]
        + [pltpu.SemaphoreType.DMA] * chunks_per_w
        + [pltpu.SemaphoreType.DMA],
    )
    def gather_kernel(table_hbm, idx_hbm, out_hbm, idx_v, rows_v, *sems):
        in_sems, out_sem = sems[:-1], sems[-1]
        wid = lax.axis_index("s") * _NC + lax.axis_index("c")
        row = wid // w_per_row
        col0 = (wid % w_per_row) * rows_per_w
        for j in range(chunks_per_w):
            pltpu.sync_copy(
                idx_hbm.at[row, pl.ds(col0 + j * _IDX_CHUNK, _IDX_CHUNK)],
                idx_v.at[j],
            )
        gathers = []
        for j in range(chunks_per_w):
            gathers.append(
                pltpu.async_copy(
                    table_hbm.at[idx_v.at[j]],
                    rows_v.at[pl.ds(j * _IDX_CHUNK, _IDX_CHUNK)],
                    in_sems[j],
                )
            )
        writes = []
        for j in range(chunks_per_w):
            gathers[j].wait()
            writes.append(
                pltpu.async_copy(
                    rows_v.at[pl.ds(j * _IDX_CHUNK, _IDX_CHUNK)],
                    out_hbm.at[pl.ds(wid * rows_per_w + j * _IDX_CHUNK, _IDX_CHUNK)],
                    out_sem,
                )
            )
        for wcopy in writes:
            wcopy.wait()

    return gather_kernel(table, idx2d)


@functools.lru_cache(maxsize=None)
def _pe_table(s, d):
    position = np.arange(s, dtype=np.float64)[:, None]
    div_term = np.exp(np.arange(0, d, 2, dtype=np.float64) * (-math.log(10000.0) / d))
    pe = np.zeros((s, d), dtype=np.float32)
    pe[:, 0::2] = np.sin(position * div_term).astype(np.float32)
    pe[:, 1::2] = np.cos(position * div_term).astype(np.float32)
    return jnp.asarray(pe)


_SBLK = 1024


def _tc_body(emb_ref, v_ref, bk_ref, pe_ref, book_ref, g_ref, beta_ref,
             out_ref, mt_ref):
    bn, s_full = v_ref.shape
    sblk, d = pe_ref.shape
    i = pl.program_id(0)

    @pl.when(i == 0)
    def _scan():
        v = v_ref[...]
        s_iota = lax.broadcasted_iota(jnp.int32, (bn, s_full), 1)
        key = jnp.where(v != 0, s_iota * 32 + v, -1)
        run = key
        sh = 1
        while sh < s_full:
            pad = jnp.full((bn, sh), -1, jnp.int32)
            run = jnp.maximum(
                run, jnp.concatenate([pad, run[:, : s_full - sh]], axis=1)
            )
            sh *= 2
        pkey = jnp.concatenate(
            [jnp.full((bn, 1), -1, jnp.int32), run[:, : s_full - 1]], axis=1
        )
        prev = jnp.where(pkey >= 0, jnp.bitwise_and(pkey, 31), -1)
        w4 = jnp.where((v != 0) & (v != prev), 1.2, 1.0).astype(jnp.float32)
        btf = bk_ref[...].astype(jnp.float32)
        m = jnp.concatenate([w4, btf], axis=0)
        mt_ref[...] = m.T

    pe = pe_ref[...]
    book = book_ref[...]
    g = g_ref[...]
    beta = beta_ref[...]
    jmat = jnp.full((d, d), 1.0 / d, dtype=jnp.float32)
    scale = math.sqrt(d)
    sl = mt_ref[pl.ds(i * sblk, sblk), :]
    for b in range(bn):
        emb = emb_ref[b]
        w = sl[:, b : b + 1]
        btc = sl[:, bn + b : bn + b + 1]
        iota4 = lax.broadcasted_iota(jnp.int32, (sblk, 4), 1).astype(jnp.float32)
        oh = (btc == iota4).astype(jnp.float32)
        z = jnp.dot(oh, book, preferred_element_type=jnp.float32)
        x = emb * scale + pe * w + z
        mu = jnp.dot(x, jmat, preferred_element_type=jnp.float32)
        m2 = jnp.dot(x * x, jmat, preferred_element_type=jnp.float32)
        r = lax.rsqrt(m2 - mu * mu + 1e-5)
        out_ref[b] = (x - mu) * r * g + beta


def kernel(token_ids, verse_positions, book_types, token_table, book_table,
           ln_gamma, ln_beta):
    bn, s = token_ids.shape
    vocab, d = token_table.shape
    n_rows = bn * s
    emb_rows = _sc_gather(token_table, token_ids.astype(jnp.int32), n_rows, d)
    emb = emb_rows.reshape(bn, s, d)

    pe = _pe_table(s, d)

    grid = s // _SBLK
    out = pl.pallas_call(
        _tc_body,
        grid=(grid,),
        in_specs=[
            pl.BlockSpec((bn, _SBLK, d), lambda i: (0, i, 0)),
            pl.BlockSpec((bn, s), lambda i: (0, 0)),
            pl.BlockSpec((bn, s), lambda i: (0, 0)),
            pl.BlockSpec((_SBLK, d), lambda i: (i, 0)),
            pl.BlockSpec((4, d), lambda i: (0, 0)),
            pl.BlockSpec((d,), lambda i: (0,)),
            pl.BlockSpec((d,), lambda i: (0,)),
        ],
        out_specs=pl.BlockSpec((bn, _SBLK, d), lambda i: (0, i, 0)),
        out_shape=jax.ShapeDtypeStruct((bn, s, d), jnp.float32),
        scratch_shapes=[pltpu.VMEM((s, 2 * bn), jnp.float32)],
    )(emb, verse_positions.astype(jnp.int32), book_types.astype(jnp.int32),
      pe, book_table.astype(jnp.float32), ln_gamma, ln_beta)
    return out


# speedup vs baseline: 1.0208x; 1.0208x over previous
import functools
import math

import jax
import jax.numpy as jnp
import numpy as np
from jax import lax
from jax.experimental import pallas as pl
from jax.experimental.pallas import tpu as pltpu
from jax.experimental.pallas import tpu_sc as plsc

_NC = 2
_NS = 16
_NW = _NC * _NS
_IDX_CHUNK = 128


def _sc_gather(table, idx2d, n_rows, d):
    chunks_per_w = idx2d.shape[0] // _NW
    rows_per_w = chunks_per_w * _IDX_CHUNK
    mesh = plsc.VectorSubcoreMesh(core_axis_name="c", subcore_axis_name="s")

    @functools.partial(
        pl.kernel,
        mesh=mesh,
        out_type=jax.ShapeDtypeStruct((n_rows, d), jnp.float32),
        scratch_types=[
            pltpu.VMEM((chunks_per_w, _IDX_CHUNK), jnp.int32),
            pltpu.VMEM((rows_per_w, d), jnp.float32),
        ]
        + [pltpu.SemaphoreType.DMA] * chunks_per_w
        + [pltpu.SemaphoreType.DMA],
    )
    def gather_kernel(table_hbm, idx_hbm, out_hbm, idx_v, rows_v, *sems):
        in_sems, out_sem = sems[:-1], sems[-1]
        wid = lax.axis_index("s") * _NC + lax.axis_index("c")
        base_chunk = wid * chunks_per_w
        pltpu.sync_copy(idx_hbm.at[pl.ds(base_chunk, chunks_per_w)], idx_v)
        gathers = []
        for j in range(chunks_per_w):
            gathers.append(
                pltpu.async_copy(
                    table_hbm.at[idx_v.at[j]],
                    rows_v.at[pl.ds(j * _IDX_CHUNK, _IDX_CHUNK)],
                    in_sems[j],
                )
            )
        writes = []
        for j in range(chunks_per_w):
            gathers[j].wait()
            writes.append(
                pltpu.async_copy(
                    rows_v.at[pl.ds(j * _IDX_CHUNK, _IDX_CHUNK)],
                    out_hbm.at[pl.ds(wid * rows_per_w + j * _IDX_CHUNK, _IDX_CHUNK)],
                    out_sem,
                )
            )
        for wcopy in writes:
            wcopy.wait()

    return gather_kernel(table, idx2d)


@functools.lru_cache(maxsize=None)
def _pe_table(s, d):
    position = np.arange(s, dtype=np.float64)[:, None]
    div_term = np.exp(np.arange(0, d, 2, dtype=np.float64) * (-math.log(10000.0) / d))
    pe = np.zeros((s, d), dtype=np.float32)
    pe[:, 0::2] = np.sin(position * div_term).astype(np.float32)
    pe[:, 1::2] = np.cos(position * div_term).astype(np.float32)
    return jnp.asarray(pe)


_SBLK = 1024


def _tc_body(emb_ref, v_ref, bk_ref, pe_ref, book_ref, g_ref, beta_ref,
             out_ref, mt_ref):
    bn, s_full = v_ref.shape
    sblk, d = pe_ref.shape
    i = pl.program_id(0)

    @pl.when(i == 0)
    def _scan():
        v = v_ref[...]
        s_iota = lax.broadcasted_iota(jnp.int32, (bn, s_full), 1)
        key = jnp.where(v != 0, s_iota * 32 + v, -1)
        run = key
        sh = 1
        while sh < s_full:
            pad = jnp.full((bn, sh), -1, jnp.int32)
            run = jnp.maximum(
                run, jnp.concatenate([pad, run[:, : s_full - sh]], axis=1)
            )
            sh *= 2
        pkey = jnp.concatenate(
            [jnp.full((bn, 1), -1, jnp.int32), run[:, : s_full - 1]], axis=1
        )
        prev = jnp.where(pkey >= 0, jnp.bitwise_and(pkey, 31), -1)
        w4 = jnp.where((v != 0) & (v != prev), 1.2, 1.0).astype(jnp.float32)
        btf = bk_ref[...].astype(jnp.float32)
        m = jnp.concatenate([w4, btf], axis=0)
        mt_ref[...] = m.T

    pe = pe_ref[...]
    book = book_ref[...]
    g = g_ref[...]
    beta = beta_ref[...]
    jmat = jnp.full((d, d), 1.0 / d, dtype=jnp.float32)
    scale = math.sqrt(d)
    sl = mt_ref[pl.ds(i * sblk, sblk), :]
    for b in range(bn):
        emb = emb_ref[b]
        w = sl[:, b : b + 1]
        btc = sl[:, bn + b : bn + b + 1]
        iota4 = lax.broadcasted_iota(jnp.int32, (sblk, 4), 1).astype(jnp.float32)
        oh = (btc == iota4).astype(jnp.float32)
        z = jnp.dot(oh, book, preferred_element_type=jnp.float32)
        x = emb * scale + pe * w + z
        mu = jnp.dot(x, jmat, preferred_element_type=jnp.float32)
        m2 = jnp.dot(x * x, jmat, preferred_element_type=jnp.float32)
        r = lax.rsqrt(m2 - mu * mu + 1e-5)
        out_ref[b] = (x - mu) * r * g + beta


def kernel(token_ids, verse_positions, book_types, token_table, book_table,
           ln_gamma, ln_beta):
    bn, s = token_ids.shape
    vocab, d = token_table.shape
    n_rows = bn * s
    idx2d = token_ids.reshape(n_rows // _IDX_CHUNK, _IDX_CHUNK).astype(jnp.int32)
    emb_rows = _sc_gather(token_table, idx2d, n_rows, d)
    emb = emb_rows.reshape(bn, s, d)

    pe = _pe_table(s, d)

    grid = s // _SBLK
    out = pl.pallas_call(
        _tc_body,
        grid=(grid,),
        in_specs=[
            pl.BlockSpec((bn, _SBLK, d), lambda i: (0, i, 0)),
            pl.BlockSpec((bn, s), lambda i: (0, 0)),
            pl.BlockSpec((bn, s), lambda i: (0, 0)),
            pl.BlockSpec((_SBLK, d), lambda i: (i, 0)),
            pl.BlockSpec((4, d), lambda i: (0, 0)),
            pl.BlockSpec((d,), lambda i: (0,)),
            pl.BlockSpec((d,), lambda i: (0,)),
        ],
        out_specs=pl.BlockSpec((bn, _SBLK, d), lambda i: (0, i, 0)),
        out_shape=jax.ShapeDtypeStruct((bn, s, d), jnp.float32),
        scratch_shapes=[pltpu.VMEM((s, 2 * bn), jnp.float32)],
    )(emb, verse_positions.astype(jnp.int32), book_types.astype(jnp.int32),
      pe, book_table.astype(jnp.float32), ln_gamma, ln_beta)
    return out


# speedup vs baseline: 1.0445x; 1.0232x over previous
import functools
import math

import jax
import jax.numpy as jnp
import numpy as np
from jax import lax
from jax.experimental import pallas as pl
from jax.experimental.pallas import tpu as pltpu
from jax.experimental.pallas import tpu_sc as plsc

_NC = 2
_NS = 16
_NW = _NC * _NS
_IDX_CHUNK = 128


def _sc_gather(table, idx2d, n_rows, d):
    chunks_per_w = idx2d.shape[0] // _NW
    rows_per_w = chunks_per_w * _IDX_CHUNK
    mesh = plsc.VectorSubcoreMesh(core_axis_name="c", subcore_axis_name="s")

    @functools.partial(
        pl.kernel,
        mesh=mesh,
        out_type=jax.ShapeDtypeStruct((n_rows, d), jnp.float32),
        scratch_types=[
            pltpu.VMEM((chunks_per_w, _IDX_CHUNK), jnp.int32),
            pltpu.VMEM((rows_per_w, d), jnp.float32),
        ]
        + [pltpu.SemaphoreType.DMA] * chunks_per_w
        + [pltpu.SemaphoreType.DMA],
    )
    def gather_kernel(table_hbm, idx_hbm, out_hbm, idx_v, rows_v, *sems):
        in_sems, out_sem = sems[:-1], sems[-1]
        wid = lax.axis_index("s") * _NC + lax.axis_index("c")
        base_chunk = wid * chunks_per_w
        pltpu.sync_copy(idx_hbm.at[pl.ds(base_chunk, chunks_per_w)], idx_v)
        gathers = []
        for j in range(chunks_per_w):
            gathers.append(
                pltpu.async_copy(
                    table_hbm.at[idx_v.at[j]],
                    rows_v.at[pl.ds(j * _IDX_CHUNK, _IDX_CHUNK)],
                    in_sems[j],
                )
            )
        writes = []
        for j in range(chunks_per_w):
            gathers[j].wait()
            writes.append(
                pltpu.async_copy(
                    rows_v.at[pl.ds(j * _IDX_CHUNK, _IDX_CHUNK)],
                    out_hbm.at[pl.ds(wid * rows_per_w + j * _IDX_CHUNK, _IDX_CHUNK)],
                    out_sem,
                )
            )
        for wcopy in writes:
            wcopy.wait()

    return gather_kernel(table, idx2d)


@functools.lru_cache(maxsize=None)
def _pe_table(s, d):
    position = np.arange(s, dtype=np.float64)[:, None]
    div_term = np.exp(np.arange(0, d, 2, dtype=np.float64) * (-math.log(10000.0) / d))
    pe = np.zeros((s, d), dtype=np.float32)
    pe[:, 0::2] = np.sin(position * div_term).astype(np.float32)
    pe[:, 1::2] = np.cos(position * div_term).astype(np.float32)
    return jnp.asarray(pe)


_SBLK = 1024


def _scan_body(v_ref, bk_ref, mt_ref):
    bn, s_full = v_ref.shape
    v = v_ref[...]
    s_iota = lax.broadcasted_iota(jnp.int32, (bn, s_full), 1)
    key = jnp.where(v != 0, s_iota * 32 + v, -1)
    run = key
    sh = 1
    while sh < s_full:
        pad = jnp.full((bn, sh), -1, jnp.int32)
        run = jnp.maximum(
            run, jnp.concatenate([pad, run[:, : s_full - sh]], axis=1)
        )
        sh *= 2
    pkey = jnp.concatenate(
        [jnp.full((bn, 1), -1, jnp.int32), run[:, : s_full - 1]], axis=1
    )
    prev = jnp.where(pkey >= 0, jnp.bitwise_and(pkey, 31), -1)
    w4 = jnp.where((v != 0) & (v != prev), 1.2, 1.0).astype(jnp.float32)
    btf = bk_ref[...].astype(jnp.float32)
    m = jnp.concatenate([w4, btf], axis=0)
    mt_ref[...] = m.T


def _tc_body(emb_ref, mt_ref, pe_ref, book_ref, g_ref, beta_ref, out_ref):
    sblk, d = pe_ref.shape
    bn = emb_ref.shape[0]
    pe = pe_ref[...]
    book = book_ref[...]
    g = g_ref[...]
    beta = beta_ref[...]
    jmat = jnp.full((d, d), 1.0 / d, dtype=jnp.float32)
    scale = math.sqrt(d)
    sl = mt_ref[...]
    iota4 = lax.broadcasted_iota(jnp.int32, (sblk, 4), 1).astype(jnp.float32)
    for b in range(bn):
        emb = emb_ref[b]
        w = sl[:, b : b + 1]
        btc = sl[:, bn + b : bn + b + 1]
        oh = (btc == iota4).astype(jnp.float32)
        z = jnp.dot(oh, book, preferred_element_type=jnp.float32)
        x = emb * scale + pe * w + z
        mu = jnp.dot(x, jmat, preferred_element_type=jnp.float32)
        m2 = jnp.dot(x * x, jmat, preferred_element_type=jnp.float32)
        r = lax.rsqrt(m2 - mu * mu + 1e-5)
        out_ref[b] = (x - mu) * r * g + beta


def kernel(token_ids, verse_positions, book_types, token_table, book_table,
           ln_gamma, ln_beta):
    bn, s = token_ids.shape
    vocab, d = token_table.shape
    n_rows = bn * s
    idx2d = token_ids.reshape(n_rows // _IDX_CHUNK, _IDX_CHUNK).astype(jnp.int32)
    emb_rows = _sc_gather(token_table, idx2d, n_rows, d)
    emb = emb_rows.reshape(bn, s, d)

    pe = _pe_table(s, d)

    mt = pl.pallas_call(
        _scan_body,
        out_shape=jax.ShapeDtypeStruct((s, 2 * bn), jnp.float32),
    )(verse_positions.astype(jnp.int32), book_types.astype(jnp.int32))

    grid = s // _SBLK
    out = pl.pallas_call(
        _tc_body,
        grid=(grid,),
        in_specs=[
            pl.BlockSpec((bn, _SBLK, d), lambda i: (0, i, 0)),
            pl.BlockSpec((_SBLK, 2 * bn), lambda i: (i, 0)),
            pl.BlockSpec((_SBLK, d), lambda i: (i, 0)),
            pl.BlockSpec((4, d), lambda i: (0, 0)),
            pl.BlockSpec((d,), lambda i: (0,)),
            pl.BlockSpec((d,), lambda i: (0,)),
        ],
        out_specs=pl.BlockSpec((bn, _SBLK, d), lambda i: (0, i, 0)),
        out_shape=jax.ShapeDtypeStruct((bn, s, d), jnp.float32),
    )(emb, mt, pe, book_table.astype(jnp.float32), ln_gamma, ln_beta)
    return out
